# baseline (speedup 1.0000x reference)
import functools

import jax
import jax.numpy as jnp
from jax import lax
from jax.experimental import pallas as pl
from jax.experimental.pallas import tpu as pltpu
from jax.experimental.pallas import tpu_sc as plsc

EMB_D = 64
SCALE = float(EMB_D) ** 0.5
LANES = 16


@functools.partial(jax.jit, static_argnames=("chunk",))
def _lookup(x_flat, table, chunk=1600):
    n_total = x_flat.shape[0]
    info = plsc.get_sparse_core_info()
    nw = info.num_cores * info.num_subcores
    b_per_w = n_total // nw
    n_chunks = b_per_w // chunk
    assert b_per_w % chunk == 0 and n_total % nw == 0

    mesh = plsc.VectorSubcoreMesh(core_axis_name="c", subcore_axis_name="s")

    @functools.partial(
        pl.kernel,
        mesh=mesh,
        out_type=jax.ShapeDtypeStruct((n_total, EMB_D), jnp.float32),
        scratch_types=[
            pltpu.VMEM((chunk,), jnp.int32),
            pltpu.VMEM((chunk, EMB_D), jnp.float32),
            pltpu.SemaphoreType.DMA,
        ],
        compiler_params=pltpu.CompilerParams(use_tc_tiling_on_sc=False),
    )
    def k(x_hbm, table_hbm, out_hbm, idx_v, rows_v, sem):
        wid = lax.axis_index("s") * info.num_cores + lax.axis_index("c")
        base = wid * b_per_w

        @pl.loop(0, n_chunks)
        def _chunk_loop(c):
            off = base + c * chunk
            pltpu.sync_copy(x_hbm.at[pl.ds(off, chunk)], idx_v)
            pltpu.async_copy(table_hbm.at[idx_v], rows_v, sem).wait()

            @pl.loop(0, chunk)
            def _scale_loop(i):
                for j in range(EMB_D // LANES):
                    sl = pl.ds(j * LANES, LANES)
                    rows_v[i, sl] = rows_v[i, sl] * SCALE

            pltpu.sync_copy(rows_v, out_hbm.at[pl.ds(off, chunk)])

    return k(x_flat, table)


def kernel(x, table):
    out = _lookup(x.reshape(-1), table)
    return out.reshape(x.shape + (EMB_D,))


# speedup vs baseline: 1.0962x; 1.0962x over previous
import functools

import jax
import jax.numpy as jnp
from jax import lax
from jax.experimental import pallas as pl
from jax.experimental.pallas import tpu as pltpu
from jax.experimental.pallas import tpu_sc as plsc

EMB_D = 64
SCALE = float(EMB_D) ** 0.5
LANES = 16
NBUF = 2


@functools.partial(jax.jit, static_argnames=("chunk",))
def _lookup(x_flat, table, chunk=800):
    n_total = x_flat.shape[0]
    info = plsc.get_sparse_core_info()
    nw = info.num_cores * info.num_subcores
    b_per_w = n_total // nw
    n_chunks = b_per_w // chunk
    assert b_per_w % chunk == 0 and n_total % nw == 0
    assert n_chunks % NBUF == 0

    mesh = plsc.VectorSubcoreMesh(core_axis_name="c", subcore_axis_name="s")

    @functools.partial(
        pl.kernel,
        mesh=mesh,
        out_type=jax.ShapeDtypeStruct((n_total, EMB_D), jnp.float32),
        scratch_types=[
            [pltpu.VMEM((chunk,), jnp.int32) for _ in range(NBUF)],
            [pltpu.VMEM((chunk, EMB_D), jnp.float32) for _ in range(NBUF)],
            [pltpu.SemaphoreType.DMA for _ in range(NBUF)],
            [pltpu.SemaphoreType.DMA for _ in range(NBUF)],
        ],
        compiler_params=pltpu.CompilerParams(use_tc_tiling_on_sc=False),
    )
    def k(x_hbm, table_hbm, out_hbm, idx_v, rows, sem_g, sem_s):
        wid = lax.axis_index("s") * info.num_cores + lax.axis_index("c")
        base = wid * b_per_w

        pltpu.sync_copy(x_hbm.at[pl.ds(base, chunk)], idx_v[0])
        pltpu.async_copy(table_hbm.at[idx_v[0]], rows[0], sem_g[0])

        @pl.loop(0, n_chunks, step=NBUF)
        def _chunk_loop(c0):
            for b in range(NBUF):
                c = c0 + b
                nb = (b + 1) % NBUF
                nxt = c + 1

                @pl.when(c > 0)
                def _wait_prev_scatter():
                    pltpu.make_async_copy(
                        rows[nb],
                        out_hbm.at[pl.ds(base + (c - 1) * chunk, chunk)],
                        sem_s[nb],
                    ).wait()

                @pl.when(nxt < n_chunks)
                def _issue_next_gather():
                    pltpu.sync_copy(
                        x_hbm.at[pl.ds(base + nxt * chunk, chunk)], idx_v[nb]
                    )
                    pltpu.async_copy(
                        table_hbm.at[idx_v[nb]], rows[nb], sem_g[nb]
                    )

                pltpu.make_async_copy(
                    table_hbm.at[idx_v[b]], rows[b], sem_g[b]
                ).wait()

                @pl.loop(0, chunk, unroll=8)
                def _scale_loop(i):
                    for j in range(EMB_D // LANES):
                        sl = pl.ds(j * LANES, LANES)
                        rows[b][i, sl] = rows[b][i, sl] * SCALE

                pltpu.async_copy(
                    rows[b],
                    out_hbm.at[pl.ds(base + c * chunk, chunk)],
                    sem_s[b],
                )

        last = n_chunks - 1
        pltpu.make_async_copy(
            rows[last % NBUF],
            out_hbm.at[pl.ds(base + last * chunk, chunk)],
            sem_s[last % NBUF],
        ).wait()

    return k(x_flat, table)


def kernel(x, table):
    out = _lookup(x.reshape(-1), table)
    return out.reshape(x.shape + (EMB_D,))


# speedup vs baseline: 1.1432x; 1.0429x over previous
import functools

import jax
import jax.numpy as jnp
from jax import lax
from jax.experimental import pallas as pl
from jax.experimental.pallas import tpu as pltpu
from jax.experimental.pallas import tpu_sc as plsc

EMB_D = 64
SCALE = float(EMB_D) ** 0.5
LANES = 16
NBUF = 2


@functools.partial(jax.jit, static_argnames=("chunk",))
def _lookup(x_flat, table, chunk=800):
    n_total = x_flat.shape[0]
    info = plsc.get_sparse_core_info()
    nw = info.num_cores * info.num_subcores
    b_per_w = n_total // nw
    n_chunks = b_per_w // chunk
    assert b_per_w % chunk == 0 and n_total % nw == 0
    assert n_chunks % NBUF == 0

    mesh = plsc.VectorSubcoreMesh(core_axis_name="c", subcore_axis_name="s")

    @functools.partial(
        pl.kernel,
        mesh=mesh,
        out_type=jax.ShapeDtypeStruct((n_total, EMB_D), jnp.float32),
        scratch_types=[
            [pltpu.VMEM((chunk,), jnp.int32) for _ in range(NBUF)],
            [pltpu.VMEM((chunk, EMB_D), jnp.float32) for _ in range(NBUF)],
            [pltpu.SemaphoreType.DMA for _ in range(NBUF)],
            [pltpu.SemaphoreType.DMA for _ in range(NBUF)],
        ],
        compiler_params=pltpu.CompilerParams(use_tc_tiling_on_sc=False),
    )
    def k(x_hbm, table_hbm, out_hbm, idx_v, rows, sem_g, sem_s):
        wid = lax.axis_index("s") * info.num_cores + lax.axis_index("c")
        base = wid * b_per_w

        pltpu.sync_copy(x_hbm.at[pl.ds(base, chunk)], idx_v[0])
        pltpu.async_copy(table_hbm.at[idx_v[0]], rows[0], sem_g[0])

        @pl.loop(0, n_chunks, step=NBUF)
        def _chunk_loop(c0):
            for b in range(NBUF):
                c = c0 + b
                nb = (b + 1) % NBUF
                nxt = c + 1

                @pl.when(c > 0)
                def _wait_prev_scatter():
                    pltpu.make_async_copy(
                        rows[nb],
                        out_hbm.at[pl.ds(base + (c - 1) * chunk, chunk)],
                        sem_s[nb],
                    ).wait()

                @pl.when(nxt < n_chunks)
                def _issue_next_gather():
                    pltpu.sync_copy(
                        x_hbm.at[pl.ds(base + nxt * chunk, chunk)], idx_v[nb]
                    )
                    pltpu.async_copy(
                        table_hbm.at[idx_v[nb]], rows[nb], sem_g[nb]
                    )

                pltpu.make_async_copy(
                    table_hbm.at[idx_v[b]], rows[b], sem_g[b]
                ).wait()

                @pl.loop(0, chunk, unroll=8)
                def _scale_loop(i):
                    for j in range(EMB_D // LANES):
                        sl = pl.ds(j * LANES, LANES)
                        rows[b][i, sl] = rows[b][i, sl] * SCALE

                pltpu.async_copy(
                    rows[b],
                    out_hbm.at[pl.ds(base + c * chunk, chunk)],
                    sem_s[b],
                )

        last = n_chunks - 1
        pltpu.make_async_copy(
            rows[last % NBUF],
            out_hbm.at[pl.ds(base + last * chunk, chunk)],
            sem_s[last % NBUF],
        ).wait()

    return k(x_flat, table)


def kernel(x, table):
    b, h = x.shape
    out = _lookup(x.T.reshape(-1), table)
    return out.reshape(h, b, EMB_D).transpose(1, 0, 2)
